# Initial kernel scaffold; baseline (speedup 1.0000x reference)
import jax
import jax.numpy as jnp
from jax.experimental import pallas as pl


def kernel(x, edge_index, W1, b1, W2, b2):
    raise NotImplementedError("write your pallas kernel here")


# speedup vs baseline: 13.9383x; 13.9383x over previous
import functools

import jax
import jax.numpy as jnp
from jax import lax
from jax.experimental import pallas as pl
from jax.experimental.pallas import tpu as pltpu
from jax.experimental.pallas import tpu_sc as plsc

_N = 10000
_NP = 10240
_E = 320000
_EP = 327680
_C = 16
_K = 10
_ALPHA = 0.1
_NT = 16
_RPT = _EP // (128 * _NT)
_NPT = _NP // _NT

_mesh = plsc.VectorSubcoreMesh(
    core_axis_name="c", subcore_axis_name="s", num_cores=1, num_subcores=_NT)
_sc_params = pltpu.CompilerParams(use_tc_tiling_on_sc=False)


def _zero_fill(buf):
    def body(i, _):
        buf[i] = jnp.zeros((16,), jnp.float32)
        return 0
    lax.fori_loop(0, buf.shape[0], body, 0)


def _deg_body(dstR, deg_out, dstbuf, ones_b, zb, acc):
    t = lax.axis_index("s")

    def fill(i, _):
        ones_b[i] = jnp.full((16,), 1.0, jnp.float32)
        zb[i] = jnp.zeros((16,), jnp.float32)
        return 0
    lax.fori_loop(0, 128, fill, 0)
    for j in range(_NPT // 128):
        pltpu.sync_copy(zb, acc.at[pl.ds(t * _NPT + j * 128, 128)])
    plsc.subcore_barrier()
    pltpu.sync_copy(dstR.at[pl.ds(t * _RPT, _RPT)], dstbuf)

    def sbody(j, _):
        pltpu.sync_copy(ones_b, acc.at[dstbuf.at[j]], add=True)
        return 0
    lax.fori_loop(0, _RPT, sbody, 0)
    plsc.subcore_barrier()
    pltpu.sync_copy(acc.at[pl.ds(t * _NPT, _NPT)],
                    deg_out.at[pl.ds(t * _NPT, _NPT)])


def _degree(dstR):
    return pl.kernel(
        _deg_body,
        out_type=jax.ShapeDtypeStruct((_NP, _C), jnp.float32),
        mesh=_mesh,
        compiler_params=_sc_params,
        scratch_types=[
            pltpu.VMEM((_RPT, 128), jnp.int32),
            pltpu.VMEM((128, _C), jnp.float32),
            pltpu.VMEM((128, _C), jnp.float32),
            pltpu.VMEM_SHARED((_NP, _C), jnp.float32),
        ],
    )(dstR)


_BLK = 512


def _prep_body(x_ref, w1_ref, b1_ref, w2_ref, b2_ref, deg_ref,
               zt0_ref, r0_ref, d9_ref, s_ref, outa_ref, dv_ref):
    i = pl.program_id(0)
    xb = x_ref[...]
    h = jnp.maximum(
        lax.dot_general(xb, w1_ref[...], (((1,), (1,)), ((), ())),
                        preferred_element_type=jnp.float32) + b1_ref[...], 0.0)
    out = lax.dot_general(h, w2_ref[...], (((1,), (1,)), ((), ())),
                          preferred_element_type=jnp.float32) + b2_ref[...]
    rows = lax.broadcasted_iota(jnp.int32, (_BLK, _C), 0) + i * _BLK
    mask = (rows < _N).astype(jnp.float32)
    out = out * mask
    dinv = lax.rsqrt(deg_ref[...] + 1.0) * mask
    d9 = 0.9 * dinv
    s = d9 * dinv
    outa = 0.1 * out
    zt0_ref[...] = dinv * out
    r0_ref[...] = s * out + outa
    d9_ref[...] = d9
    s_ref[...] = s
    outa_ref[...] = outa
    dv_ref[...] = dinv


def _prep(x_p, W1, b1, W2, b2, degrows):
    o16 = jax.ShapeDtypeStruct((_NP, _C), jnp.float32)
    bs16 = pl.BlockSpec((_BLK, _C), lambda i: (i, 0))
    return pl.pallas_call(
        _prep_body,
        grid=(_NP // _BLK,),
        in_specs=[
            pl.BlockSpec((_BLK, 128), lambda i: (i, 0)),
            pl.BlockSpec((128, 128), lambda i: (0, 0)),
            pl.BlockSpec((1, 128), lambda i: (0, 0)),
            pl.BlockSpec((_C, 128), lambda i: (0, 0)),
            pl.BlockSpec((1, _C), lambda i: (0, 0)),
            bs16,
        ],
        out_specs=[bs16] * 6,
        out_shape=[o16] * 6,
    )(x_p, W1, b1.reshape(1, 128), W2, b2.reshape(1, _C), degrows)


def _prop_body(zt0, r0, d9h, sh, outah, dvh, srcR, dstR,
               z_out, ztA, ztB,
               srcbuf, dstbuf, rows0, zb, d9b, sb, outab, dvb, rb, accv, acc):
    t = lax.axis_index("s")
    sl = pl.ds(t * _NPT, _NPT)
    pltpu.sync_copy(srcR.at[pl.ds(t * _RPT, _RPT)], srcbuf)
    pltpu.sync_copy(dstR.at[pl.ds(t * _RPT, _RPT)], dstbuf)
    pltpu.sync_copy(d9h.at[sl], d9b)
    pltpu.sync_copy(sh.at[sl], sb)
    pltpu.sync_copy(outah.at[sl], outab)
    pltpu.sync_copy(dvh.at[sl], dvb)
    pltpu.sync_copy(r0.at[sl], rb)
    _zero_fill(zb)
    for j in range(_NPT // 128):
        pltpu.sync_copy(zb, acc.at[pl.ds(t * _NPT + j * 128, 128)])
    plsc.subcore_barrier()

    for k in range(_K):
        zt_src = zt0 if k == 0 else (ztA if (k % 2) == 1 else ztB)
        zt_dst = ztA if (k % 2) == 0 else ztB
        last = k == _K - 1

        def sbody(j, _):
            pltpu.sync_copy(zt_src.at[srcbuf.at[j]], rows0)
            pltpu.sync_copy(rows0, acc.at[dstbuf.at[j]], add=True)
            return 0
        lax.fori_loop(0, _RPT, sbody, 0)
        plsc.subcore_barrier()

        pltpu.sync_copy(acc.at[sl], accv)

        def cbody(i, _):
            a = accv[i]
            z16 = d9b[i] * a + rb[i]
            accv[i] = z16 if last else dvb[i] * z16
            rb[i] = sb[i] * z16 + outab[i]
            return 0
        lax.fori_loop(0, _NPT, cbody, 0)
        if last:
            pltpu.sync_copy(accv, z_out.at[sl])
        else:
            pltpu.sync_copy(accv, zt_dst.at[sl])
            for j in range(_NPT // 128):
                pltpu.sync_copy(zb, acc.at[pl.ds(t * _NPT + j * 128, 128)])
        plsc.subcore_barrier()


def _propagate(zt0, r0, d9, s, outa, dv, srcR, dstR):
    o16 = jax.ShapeDtypeStruct((_NP, _C), jnp.float32)
    z, _, _ = pl.kernel(
        _prop_body,
        out_type=(o16, o16, o16),
        mesh=_mesh,
        compiler_params=_sc_params,
        scratch_types=[
            pltpu.VMEM((_RPT, 128), jnp.int32),
            pltpu.VMEM((_RPT, 128), jnp.int32),
            pltpu.VMEM((128, _C), jnp.float32),
            pltpu.VMEM((128, _C), jnp.float32),
            pltpu.VMEM((_NPT, _C), jnp.float32),
            pltpu.VMEM((_NPT, _C), jnp.float32),
            pltpu.VMEM((_NPT, _C), jnp.float32),
            pltpu.VMEM((_NPT, _C), jnp.float32),
            pltpu.VMEM((_NPT, _C), jnp.float32),
            pltpu.VMEM((_NPT, _C), jnp.float32),
            pltpu.VMEM_SHARED((_NP, _C), jnp.float32),
        ],
    )(zt0, r0, d9, s, outa, dv, srcR, dstR)
    return z


def kernel(x, edge_index, W1, b1, W2, b2):
    pad = jnp.full((_EP - _E,), _N, jnp.int32)
    srcR = jnp.concatenate([edge_index[0], pad]).reshape(_EP // 128, 128)
    dstR = jnp.concatenate([edge_index[1], pad]).reshape(_EP // 128, 128)
    x_p = jnp.pad(x, ((0, _NP - _N), (0, 0)))
    degrows = _degree(dstR)
    zt0, r0, d9, s, outa, dv = _prep(x_p, W1, b1, W2, b2, degrows)
    z = _propagate(zt0, r0, d9, s, outa, dv, srcR, dstR)
    return z[:_N]


# speedup vs baseline: 23.4676x; 1.6837x over previous
import functools

import jax
import jax.numpy as jnp
from jax import lax
from jax.experimental import pallas as pl
from jax.experimental.pallas import tpu as pltpu
from jax.experimental.pallas import tpu_sc as plsc

_N = 10000
_NP = 10240
_E = 320000
_EP = 327680
_C = 16
_K = 10
_ALPHA = 0.1
_NT = 16
_RPT = _EP // (128 * _NT)
_NPT = _NP // _NT
_EPT = _EP // _NT
_CH = 1024

_mesh = plsc.VectorSubcoreMesh(
    core_axis_name="c", subcore_axis_name="s", num_cores=1, num_subcores=_NT)
_sc_params = pltpu.CompilerParams(use_tc_tiling_on_sc=False)


def _zero_fill(buf):
    def body(i, _):
        buf[i] = jnp.zeros((16,), jnp.float32)
        return 0
    lax.fori_loop(0, buf.shape[0], body, 0)


def _deg_body(dstR, deg_out, dstbuf, ones_b, zb, acc):
    t = lax.axis_index("s")

    def fill(i, _):
        ones_b[i] = jnp.full((16,), 1.0, jnp.float32)
        return 0
    lax.fori_loop(0, _CH, fill, 0)
    _zero_fill(zb)
    for j in range(_NPT // 128):
        pltpu.sync_copy(zb, acc.at[pl.ds(t * _NPT + j * 128, 128)])
    plsc.subcore_barrier()
    pltpu.sync_copy(dstR.at[pl.ds(t * _EPT, _EPT)], dstbuf)

    def sbody(j, _):
        pltpu.sync_copy(ones_b, acc.at[dstbuf.at[pl.ds(j * _CH, _CH)]],
                        add=True)
        return 0
    lax.fori_loop(0, _EPT // _CH, sbody, 0)
    plsc.subcore_barrier()
    pltpu.sync_copy(acc.at[pl.ds(t * _NPT, _NPT)],
                    deg_out.at[pl.ds(t * _NPT, _NPT)])


def _degree(dstR):
    return pl.kernel(
        _deg_body,
        out_type=jax.ShapeDtypeStruct((_NP, _C), jnp.float32),
        mesh=_mesh,
        compiler_params=_sc_params,
        scratch_types=[
            pltpu.VMEM((_EPT,), jnp.int32),
            pltpu.VMEM((_CH, _C), jnp.float32),
            pltpu.VMEM((128, _C), jnp.float32),
            pltpu.VMEM_SHARED((_NP, _C), jnp.float32),
        ],
    )(dstR)


_BLK = 512


def _prep_body(x_ref, w1_ref, b1_ref, w2_ref, b2_ref, deg_ref,
               zt0_ref, r0_ref, d9_ref, s_ref, outa_ref, dv_ref):
    i = pl.program_id(0)
    xb = x_ref[...]
    h = jnp.maximum(
        lax.dot_general(xb, w1_ref[...], (((1,), (1,)), ((), ())),
                        preferred_element_type=jnp.float32) + b1_ref[...], 0.0)
    out = lax.dot_general(h, w2_ref[...], (((1,), (1,)), ((), ())),
                          preferred_element_type=jnp.float32) + b2_ref[...]
    rows = lax.broadcasted_iota(jnp.int32, (_BLK, _C), 0) + i * _BLK
    mask = (rows < _N).astype(jnp.float32)
    out = out * mask
    dinv = lax.rsqrt(deg_ref[...] + 1.0) * mask
    d9 = 0.9 * dinv
    s = d9 * dinv
    outa = 0.1 * out
    zt0_ref[...] = dinv * out
    r0_ref[...] = s * out + outa
    d9_ref[...] = d9
    s_ref[...] = s
    outa_ref[...] = outa
    dv_ref[...] = dinv


def _prep(x_p, W1, b1, W2, b2, degrows):
    o16 = jax.ShapeDtypeStruct((_NP, _C), jnp.float32)
    bs16 = pl.BlockSpec((_BLK, _C), lambda i: (i, 0))
    return pl.pallas_call(
        _prep_body,
        grid=(_NP // _BLK,),
        in_specs=[
            pl.BlockSpec((_BLK, 128), lambda i: (i, 0)),
            pl.BlockSpec((128, 128), lambda i: (0, 0)),
            pl.BlockSpec((1, 128), lambda i: (0, 0)),
            pl.BlockSpec((_C, 128), lambda i: (0, 0)),
            pl.BlockSpec((1, _C), lambda i: (0, 0)),
            bs16,
        ],
        out_specs=[bs16] * 6,
        out_shape=[o16] * 6,
    )(x_p, W1, b1.reshape(1, 128), W2, b2.reshape(1, _C), degrows)


def _prop_body(zt0, r0, d9h, sh, outah, dvh, srcR, dstR,
               z_out, ztA, ztB,
               srcbuf, dstbuf, rows0, zb, d9b, sb, outab, dvb, rb, accv, acc):
    t = lax.axis_index("s")
    sl = pl.ds(t * _NPT, _NPT)
    pltpu.sync_copy(srcR.at[pl.ds(t * _EPT, _EPT)], srcbuf)
    pltpu.sync_copy(dstR.at[pl.ds(t * _EPT, _EPT)], dstbuf)
    pltpu.sync_copy(d9h.at[sl], d9b)
    pltpu.sync_copy(sh.at[sl], sb)
    pltpu.sync_copy(outah.at[sl], outab)
    pltpu.sync_copy(dvh.at[sl], dvb)
    pltpu.sync_copy(r0.at[sl], rb)
    _zero_fill(zb)
    for j in range(_NPT // 128):
        pltpu.sync_copy(zb, acc.at[pl.ds(t * _NPT + j * 128, 128)])
    plsc.subcore_barrier()

    for k in range(_K):
        zt_src = zt0 if k == 0 else (ztA if (k % 2) == 1 else ztB)
        zt_dst = ztA if (k % 2) == 0 else ztB
        last = k == _K - 1

        def sbody(j, _):
            pltpu.sync_copy(zt_src.at[srcbuf.at[pl.ds(j * _CH, _CH)]], rows0)
            pltpu.sync_copy(rows0, acc.at[dstbuf.at[pl.ds(j * _CH, _CH)]],
                            add=True)
            return 0
        lax.fori_loop(0, _EPT // _CH, sbody, 0)
        plsc.subcore_barrier()

        pltpu.sync_copy(acc.at[sl], accv)

        def cbody(i, _):
            a = accv[i]
            z16 = d9b[i] * a + rb[i]
            accv[i] = z16 if last else dvb[i] * z16
            rb[i] = sb[i] * z16 + outab[i]
            return 0
        lax.fori_loop(0, _NPT, cbody, 0)
        if last:
            pltpu.sync_copy(accv, z_out.at[sl])
        else:
            pltpu.sync_copy(accv, zt_dst.at[sl])
            for j in range(_NPT // 128):
                pltpu.sync_copy(zb, acc.at[pl.ds(t * _NPT + j * 128, 128)])
        plsc.subcore_barrier()


def _propagate(zt0, r0, d9, s, outa, dv, srcR, dstR):
    o16 = jax.ShapeDtypeStruct((_NP, _C), jnp.float32)
    z, _, _ = pl.kernel(
        _prop_body,
        out_type=(o16, o16, o16),
        mesh=_mesh,
        compiler_params=_sc_params,
        scratch_types=[
            pltpu.VMEM((_EPT,), jnp.int32),
            pltpu.VMEM((_EPT,), jnp.int32),
            pltpu.VMEM((_CH, _C), jnp.float32),
            pltpu.VMEM((128, _C), jnp.float32),
            pltpu.VMEM((_NPT, _C), jnp.float32),
            pltpu.VMEM((_NPT, _C), jnp.float32),
            pltpu.VMEM((_NPT, _C), jnp.float32),
            pltpu.VMEM((_NPT, _C), jnp.float32),
            pltpu.VMEM((_NPT, _C), jnp.float32),
            pltpu.VMEM((_NPT, _C), jnp.float32),
            pltpu.VMEM_SHARED((_NP, _C), jnp.float32),
        ],
    )(zt0, r0, d9, s, outa, dv, srcR, dstR)
    return z


def kernel(x, edge_index, W1, b1, W2, b2):
    pad = jnp.full((_EP - _E,), _N, jnp.int32)
    srcR = jnp.concatenate([edge_index[0], pad])
    dstR = jnp.concatenate([edge_index[1], pad])
    x_p = jnp.pad(x, ((0, _NP - _N), (0, 0)))
    degrows = _degree(dstR)
    zt0, r0, d9, s, outa, dv = _prep(x_p, W1, b1, W2, b2, degrows)
    z = _propagate(zt0, r0, d9, s, outa, dv, srcR, dstR)
    return z[:_N]


# speedup vs baseline: 25.3764x; 1.0813x over previous
import functools

import jax
import jax.numpy as jnp
from jax import lax
from jax.experimental import pallas as pl
from jax.experimental.pallas import tpu as pltpu
from jax.experimental.pallas import tpu_sc as plsc

_N = 10000
_NP = 10240
_E = 320000
_EP = 327680
_C = 16
_K = 10
_ALPHA = 0.1
_NT = 16
_RPT = _EP // (128 * _NT)
_NPT = _NP // _NT
_EPT = _EP // _NT
_CH = 1024

_mesh = plsc.VectorSubcoreMesh(
    core_axis_name="c", subcore_axis_name="s", num_cores=1, num_subcores=_NT)
_sc_params = pltpu.CompilerParams(use_tc_tiling_on_sc=False)


def _zero_fill(buf):
    def body(i, _):
        buf[i] = jnp.zeros((16,), jnp.float32)
        return 0
    lax.fori_loop(0, buf.shape[0], body, 0)


def _deg_body(dstR, deg_out, dstbuf, ones_b, zb, acc):
    t = lax.axis_index("s")

    def fill(i, _):
        ones_b[i] = jnp.full((16,), 1.0, jnp.float32)
        return 0
    lax.fori_loop(0, _CH, fill, 0)
    _zero_fill(zb)
    for j in range(_NPT // 128):
        pltpu.sync_copy(zb, acc.at[pl.ds(t * _NPT + j * 128, 128)])
    plsc.subcore_barrier()
    pltpu.sync_copy(dstR.at[pl.ds(t * _EPT, _EPT)], dstbuf)

    def sbody(j, _):
        pltpu.sync_copy(ones_b, acc.at[dstbuf.at[pl.ds(j * _CH, _CH)]],
                        add=True)
        return 0
    lax.fori_loop(0, _EPT // _CH, sbody, 0)
    plsc.subcore_barrier()
    pltpu.sync_copy(acc.at[pl.ds(t * _NPT, _NPT)],
                    deg_out.at[pl.ds(t * _NPT, _NPT)])


def _degree(dstR):
    return pl.kernel(
        _deg_body,
        out_type=jax.ShapeDtypeStruct((_NP, _C), jnp.float32),
        mesh=_mesh,
        compiler_params=_sc_params,
        scratch_types=[
            pltpu.VMEM((_EPT,), jnp.int32),
            pltpu.VMEM((_CH, _C), jnp.float32),
            pltpu.VMEM((128, _C), jnp.float32),
            pltpu.VMEM_SHARED((_NP, _C), jnp.float32),
        ],
    )(dstR)


_BLK = 512


def _prep_body(x_ref, w1_ref, b1_ref, w2_ref, b2_ref, deg_ref,
               zt0_ref, r0_ref, outa_ref, dv_ref):
    i = pl.program_id(0)
    xb = x_ref[...]
    h = jnp.maximum(
        lax.dot_general(xb, w1_ref[...], (((1,), (1,)), ((), ())),
                        preferred_element_type=jnp.float32) + b1_ref[...], 0.0)
    out = lax.dot_general(h, w2_ref[...], (((1,), (1,)), ((), ())),
                          preferred_element_type=jnp.float32) + b2_ref[...]
    rows = lax.broadcasted_iota(jnp.int32, (_BLK, _C), 0) + i * _BLK
    mask = (rows < _N).astype(jnp.float32)
    out = out * mask
    dinv = lax.rsqrt(deg_ref[...] + 1.0) * mask
    s = 0.9 * dinv * dinv
    outa = 0.1 * out
    zt0_ref[...] = dinv * out
    r0_ref[...] = s * out + outa
    outa_ref[...] = outa
    dv_ref[...] = dinv


def _prep(x_p, W1, b1, W2, b2, degrows):
    o16 = jax.ShapeDtypeStruct((_NP, _C), jnp.float32)
    bs16 = pl.BlockSpec((_BLK, _C), lambda i: (i, 0))
    return pl.pallas_call(
        _prep_body,
        grid=(_NP // _BLK,),
        in_specs=[
            pl.BlockSpec((_BLK, 128), lambda i: (i, 0)),
            pl.BlockSpec((128, 128), lambda i: (0, 0)),
            pl.BlockSpec((1, 128), lambda i: (0, 0)),
            pl.BlockSpec((_C, 128), lambda i: (0, 0)),
            pl.BlockSpec((1, _C), lambda i: (0, 0)),
            bs16,
        ],
        out_specs=[bs16] * 4,
        out_shape=[o16] * 4,
    )(x_p, W1, b1.reshape(1, 128), W2, b2.reshape(1, _C), degrows)


def _prop_body(zt0, r0, outah, dvh, srcR, dstR,
               z_out, ztA, ztB,
               srcbuf, dstbuf, rows0, rows1, zb, outab, dvb, rb, accv, acc,
               sg0, sg1, ss0, ss1):
    t = lax.axis_index("s")
    sl = pl.ds(t * _NPT, _NPT)
    pltpu.sync_copy(srcR.at[pl.ds(t * _EPT, _EPT)], srcbuf)
    pltpu.sync_copy(dstR.at[pl.ds(t * _EPT, _EPT)], dstbuf)
    pltpu.sync_copy(outah.at[sl], outab)
    pltpu.sync_copy(dvh.at[sl], dvb)
    pltpu.sync_copy(r0.at[sl], rb)
    _zero_fill(zb)
    for j in range(_NPT // 128):
        pltpu.sync_copy(zb, acc.at[pl.ds(t * _NPT + j * 128, 128)])
    plsc.subcore_barrier()

    nch = _EPT // _CH
    bufs = (rows0, rows1)
    gsems = (sg0, sg1)
    ssems = (ss0, ss1)

    def gather(k, j):
        zt_src = zt0 if k == 0 else (ztA if (k % 2) == 1 else ztB)
        return pltpu.async_copy(
            zt_src.at[srcbuf.at[pl.ds(j * _CH, _CH)]], bufs[j % 2],
            gsems[j % 2])

    def scat(j):
        return pltpu.async_copy(
            bufs[j % 2], acc.at[dstbuf.at[pl.ds(j * _CH, _CH)]], ssems[j % 2],
            add=True)

    for k in range(_K):
        zt_dst = ztA if (k % 2) == 0 else ztB
        last = k == _K - 1

        g = gather(k, 0)
        s_prev = None
        s_prev2 = None
        for j in range(nch):
            g.wait()
            s = scat(j)
            if j + 1 < nch:
                if s_prev is not None:
                    s_prev.wait()
                g = gather(k, j + 1)
            s_prev2 = s_prev
            s_prev = s
        s_prev2.wait()
        s_prev.wait()
        plsc.subcore_barrier()

        pltpu.sync_copy(acc.at[sl], accv)

        def cbody(i, _):
            a = accv[i]
            z16 = 0.9 * (dvb[i] * a) + rb[i]
            zt16 = dvb[i] * z16
            accv[i] = z16 if last else zt16
            rb[i] = 0.9 * (dvb[i] * zt16) + outab[i]
            return 0
        lax.fori_loop(0, _NPT, cbody, 0)
        if last:
            pltpu.sync_copy(accv, z_out.at[sl])
        else:
            pltpu.sync_copy(accv, zt_dst.at[sl])
            for j in range(_NPT // 128):
                pltpu.sync_copy(zb, acc.at[pl.ds(t * _NPT + j * 128, 128)])
        plsc.subcore_barrier()


def _propagate(zt0, r0, outa, dv, srcR, dstR):
    o16 = jax.ShapeDtypeStruct((_NP, _C), jnp.float32)
    z, _, _ = pl.kernel(
        _prop_body,
        out_type=(o16, o16, o16),
        mesh=_mesh,
        compiler_params=_sc_params,
        scratch_types=[
            pltpu.VMEM((_EPT,), jnp.int32),
            pltpu.VMEM((_EPT,), jnp.int32),
            pltpu.VMEM((_CH, _C), jnp.float32),
            pltpu.VMEM((_CH, _C), jnp.float32),
            pltpu.VMEM((128, _C), jnp.float32),
            pltpu.VMEM((_NPT, _C), jnp.float32),
            pltpu.VMEM((_NPT, _C), jnp.float32),
            pltpu.VMEM((_NPT, _C), jnp.float32),
            pltpu.VMEM((_NPT, _C), jnp.float32),
            pltpu.VMEM_SHARED((_NP, _C), jnp.float32),
            pltpu.SemaphoreType.DMA,
            pltpu.SemaphoreType.DMA,
            pltpu.SemaphoreType.DMA,
            pltpu.SemaphoreType.DMA,
        ],
    )(zt0, r0, outa, dv, srcR, dstR)
    return z


def kernel(x, edge_index, W1, b1, W2, b2):
    pad = jnp.full((_EP - _E,), _N, jnp.int32)
    srcR = jnp.concatenate([edge_index[0], pad])
    dstR = jnp.concatenate([edge_index[1], pad])
    x_p = jnp.pad(x, ((0, _NP - _N), (0, 0)))
    degrows = _degree(dstR)
    zt0, r0, outa, dv = _prep(x_p, W1, b1, W2, b2, degrows)
    z = _propagate(zt0, r0, outa, dv, srcR, dstR)
    return z[:_N]


# speedup vs baseline: 45.4328x; 1.7904x over previous
import jax
import jax.numpy as jnp
from jax import lax
from jax.experimental import pallas as pl
from jax.experimental.pallas import tpu as pltpu
from jax.experimental.pallas import tpu_sc as plsc

_N = 10000
_NP = 10240
_E = 320000
_EP = 327680
_C = 16
_K = 10
_NT = 16
_NPT = _NP // _NT
_EPT = _EP // _NT
_CH = 512

_mesh = plsc.VectorSubcoreMesh(
    core_axis_name="c", subcore_axis_name="s", num_cores=1, num_subcores=_NT)
_sc_params = pltpu.CompilerParams(use_tc_tiling_on_sc=False)


def _zero_fill(buf):
    def body(i, _):
        buf[i] = jnp.zeros((16,), jnp.float32)
        return 0
    lax.fori_loop(0, buf.shape[0], body, 0)


def _split_packed(packed_buf, dst_buf):
    def body(i, _):
        v = packed_buf[pl.ds(i * 16, 16)]
        dst_buf[pl.ds(i * 16, 16)] = lax.bitwise_and(v, 16383)
        packed_buf[pl.ds(i * 16, 16)] = lax.shift_right_logical(v, 14)
        return 0
    lax.fori_loop(0, _EPT // 16, body, 0)


def _deg_body(edges, deg_out, ebuf, dstbuf, ones_b, zb, acc):
    t = lax.axis_index("s")

    def fill(i, _):
        ones_b[i] = jnp.full((16,), 1.0, jnp.float32)
        return 0
    lax.fori_loop(0, _CH, fill, 0)
    _zero_fill(zb)
    for j in range(_NPT // 128):
        pltpu.sync_copy(zb, acc.at[pl.ds(t * _NPT + j * 128, 128)])
    pltpu.sync_copy(edges.at[pl.ds(t * _EPT, _EPT)], ebuf)
    _split_packed(ebuf, dstbuf)
    plsc.subcore_barrier()

    def sbody(j, _):
        pltpu.sync_copy(ones_b, acc.at[dstbuf.at[pl.ds(j * _CH, _CH)]],
                        add=True)
        return 0
    lax.fori_loop(0, _EPT // _CH, sbody, 0)
    plsc.subcore_barrier()
    pltpu.sync_copy(acc.at[pl.ds(t * _NPT, _NPT)],
                    deg_out.at[pl.ds(t * _NPT, _NPT)])


def _degree(edges):
    return pl.kernel(
        _deg_body,
        out_type=jax.ShapeDtypeStruct((_NP, _C), jnp.float32),
        mesh=_mesh,
        compiler_params=_sc_params,
        scratch_types=[
            pltpu.VMEM((_EPT,), jnp.int32),
            pltpu.VMEM((_EPT,), jnp.int32),
            pltpu.VMEM((_CH, _C), jnp.float32),
            pltpu.VMEM((128, _C), jnp.float32),
            pltpu.VMEM_SHARED((_NP, _C), jnp.float32),
        ],
    )(edges)


_BLK = 512


def _prep_body(x_ref, w1_ref, b1_ref, w2_ref, b2_ref, deg_ref,
               zt0_ref, r0_ref, outa_ref, dv_ref):
    i = pl.program_id(0)
    xb = x_ref[...]
    h = jnp.maximum(
        lax.dot_general(xb, w1_ref[...], (((1,), (1,)), ((), ())),
                        preferred_element_type=jnp.float32) + b1_ref[...], 0.0)
    out = lax.dot_general(h, w2_ref[...], (((1,), (1,)), ((), ())),
                          preferred_element_type=jnp.float32) + b2_ref[...]
    rows = lax.broadcasted_iota(jnp.int32, (_BLK, _C), 0) + i * _BLK
    mask = (rows < _N).astype(jnp.float32)
    out = out * mask
    dinv = lax.rsqrt(deg_ref[...] + 1.0) * mask
    s = 0.9 * dinv * dinv
    outa = 0.1 * out
    zt0_ref[...] = dinv * out
    r0_ref[...] = s * out + outa
    outa_ref[...] = outa
    dv_ref[...] = dinv


def _prep(x_p, W1, b1, W2, b2, degrows):
    o16 = jax.ShapeDtypeStruct((_NP, _C), jnp.float32)
    bs16 = pl.BlockSpec((_BLK, _C), lambda i: (i, 0))
    return pl.pallas_call(
        _prep_body,
        grid=(_NP // _BLK,),
        in_specs=[
            pl.BlockSpec((_BLK, 128), lambda i: (i, 0)),
            pl.BlockSpec((128, 128), lambda i: (0, 0)),
            pl.BlockSpec((1, 128), lambda i: (0, 0)),
            pl.BlockSpec((_C, 128), lambda i: (0, 0)),
            pl.BlockSpec((1, _C), lambda i: (0, 0)),
            bs16,
        ],
        out_specs=[bs16] * 4,
        out_shape=[o16] * 4,
    )(x_p, W1, b1.reshape(1, 128), W2, b2.reshape(1, _C), degrows)


def _prop_body(zt0, r0, outah, dvh, edges,
               z_out,
               srcbuf, dstbuf, rows0, rows1, zb, outab, dvb, rb, accv,
               acc, ztS,
               sg0, sg1, ss0, ss1):
    t = lax.axis_index("s")
    sl = pl.ds(t * _NPT, _NPT)
    pltpu.sync_copy(edges.at[pl.ds(t * _EPT, _EPT)], srcbuf)
    _split_packed(srcbuf, dstbuf)
    pltpu.sync_copy(outah.at[sl], outab)
    pltpu.sync_copy(dvh.at[sl], dvb)
    pltpu.sync_copy(r0.at[sl], rb)
    pltpu.sync_copy(zt0.at[sl], ztS.at[sl])
    _zero_fill(zb)
    for j in range(_NPT // 128):
        pltpu.sync_copy(zb, acc.at[pl.ds(t * _NPT + j * 128, 128)])
    plsc.subcore_barrier()

    nch = _EPT // _CH
    bufs = (rows0, rows1)
    gsems = (sg0, sg1)
    ssems = (ss0, ss1)

    def gather(j):
        return pltpu.async_copy(
            ztS.at[srcbuf.at[pl.ds(j * _CH, _CH)]], bufs[j % 2],
            gsems[j % 2])

    def scat(j):
        return pltpu.async_copy(
            bufs[j % 2], acc.at[dstbuf.at[pl.ds(j * _CH, _CH)]], ssems[j % 2],
            add=True)

    for k in range(_K):
        last = k == _K - 1

        g = gather(0)
        s_prev = None
        s_prev2 = None
        for j in range(nch):
            g.wait()
            s = scat(j)
            if j + 1 < nch:
                if s_prev is not None:
                    s_prev.wait()
                g = gather(j + 1)
            s_prev2 = s_prev
            s_prev = s
        s_prev2.wait()
        s_prev.wait()
        plsc.subcore_barrier()

        pltpu.sync_copy(acc.at[sl], accv)

        def cbody(i, _):
            a = accv[i]
            z16 = 0.9 * (dvb[i] * a) + rb[i]
            zt16 = dvb[i] * z16
            accv[i] = z16 if last else zt16
            rb[i] = 0.9 * (dvb[i] * zt16) + outab[i]
            return 0
        lax.fori_loop(0, _NPT, cbody, 0)
        if last:
            pltpu.sync_copy(accv, z_out.at[sl])
        else:
            pltpu.sync_copy(accv, ztS.at[sl])
            for j in range(_NPT // 128):
                pltpu.sync_copy(zb, acc.at[pl.ds(t * _NPT + j * 128, 128)])
        plsc.subcore_barrier()


def _propagate(zt0, r0, outa, dv, edges):
    o16 = jax.ShapeDtypeStruct((_NP, _C), jnp.float32)
    z = pl.kernel(
        _prop_body,
        out_type=o16,
        mesh=_mesh,
        compiler_params=_sc_params,
        scratch_types=[
            pltpu.VMEM((_EPT,), jnp.int32),
            pltpu.VMEM((_EPT,), jnp.int32),
            pltpu.VMEM((_CH, _C), jnp.float32),
            pltpu.VMEM((_CH, _C), jnp.float32),
            pltpu.VMEM((128, _C), jnp.float32),
            pltpu.VMEM((_NPT, _C), jnp.float32),
            pltpu.VMEM((_NPT, _C), jnp.float32),
            pltpu.VMEM((_NPT, _C), jnp.float32),
            pltpu.VMEM((_NPT, _C), jnp.float32),
            pltpu.VMEM_SHARED((_NP, _C), jnp.float32),
            pltpu.VMEM_SHARED((_NP, _C), jnp.float32),
            pltpu.SemaphoreType.DMA,
            pltpu.SemaphoreType.DMA,
            pltpu.SemaphoreType.DMA,
            pltpu.SemaphoreType.DMA,
        ],
    )(zt0, r0, outa, dv, edges)
    return z


def kernel(x, edge_index, W1, b1, W2, b2):
    pad = jnp.full((_EP - _E,), _N, jnp.int32)
    src = jnp.concatenate([edge_index[0], pad])
    dst = jnp.concatenate([edge_index[1], pad])
    edges = jnp.left_shift(src, 14) | dst
    x_p = jnp.pad(x, ((0, _NP - _N), (0, 0)))
    degrows = _degree(edges)
    zt0, r0, outa, dv = _prep(x_p, W1, b1, W2, b2, degrows)
    z = _propagate(zt0, r0, outa, dv, edges)
    return z[:_N]


# speedup vs baseline: 48.5908x; 1.0695x over previous
import jax
import jax.numpy as jnp
from jax import lax
from jax.experimental import pallas as pl
from jax.experimental.pallas import tpu as pltpu
from jax.experimental.pallas import tpu_sc as plsc

_N = 10000
_NP = 10240
_E = 320000
_EP = 327680
_C = 16
_K = 10
_NT = 16
_NPT = _NP // _NT
_EPT = _EP // _NT
_CH = 512

_mesh = plsc.VectorSubcoreMesh(
    core_axis_name="c", subcore_axis_name="s", num_cores=1, num_subcores=_NT)
_sc_params = pltpu.CompilerParams(use_tc_tiling_on_sc=False)


def _zero_fill(buf):
    def body(i, _):
        buf[i] = jnp.zeros((16,), jnp.float32)
        return 0
    lax.fori_loop(0, buf.shape[0], body, 0)


def _split_packed(packed_buf, dst_buf):
    def body(i, _):
        v = packed_buf[pl.ds(i * 16, 16)]
        dst_buf[pl.ds(i * 16, 16)] = lax.bitwise_and(v, 16383)
        packed_buf[pl.ds(i * 16, 16)] = lax.shift_right_logical(v, 14)
        return 0
    lax.fori_loop(0, _EPT // 16, body, 0)


def _deg_body(edges, deg_out, ebuf, dstbuf, ones_b, zb, acc):
    t = lax.axis_index("s")

    def fill(i, _):
        ones_b[i] = jnp.full((16,), 1.0, jnp.float32)
        return 0
    lax.fori_loop(0, _CH, fill, 0)
    _zero_fill(zb)
    for j in range(_NPT // 128):
        pltpu.sync_copy(zb, acc.at[pl.ds(t * _NPT + j * 128, 128)])
    pltpu.sync_copy(edges.at[pl.ds(t * _EPT, _EPT)], ebuf)
    _split_packed(ebuf, dstbuf)
    plsc.subcore_barrier()

    def sbody(j, _):
        pltpu.sync_copy(ones_b, acc.at[dstbuf.at[pl.ds(j * _CH, _CH)]],
                        add=True)
        return 0
    lax.fori_loop(0, _EPT // _CH, sbody, 0)
    plsc.subcore_barrier()
    pltpu.sync_copy(acc.at[pl.ds(t * _NPT, _NPT)],
                    deg_out.at[pl.ds(t * _NPT, _NPT)])


def _degree(edges):
    return pl.kernel(
        _deg_body,
        out_type=jax.ShapeDtypeStruct((_NP, _C), jnp.float32),
        mesh=_mesh,
        compiler_params=_sc_params,
        scratch_types=[
            pltpu.VMEM((_EPT,), jnp.int32),
            pltpu.VMEM((_EPT,), jnp.int32),
            pltpu.VMEM((_CH, _C), jnp.float32),
            pltpu.VMEM((128, _C), jnp.float32),
            pltpu.VMEM_SHARED((_NP, _C), jnp.float32),
        ],
    )(edges)


_BLK = 512


def _prep_body(x_ref, w1_ref, b1_ref, w2_ref, b2_ref, deg_ref,
               zt0_ref, r0_ref, outa_ref, dv_ref):
    i = pl.program_id(0)
    xb = x_ref[...]
    h = jnp.maximum(
        lax.dot_general(xb, w1_ref[...], (((1,), (1,)), ((), ())),
                        preferred_element_type=jnp.float32) + b1_ref[...], 0.0)
    out = lax.dot_general(h, w2_ref[...], (((1,), (1,)), ((), ())),
                          preferred_element_type=jnp.float32) + b2_ref[...]
    rows = lax.broadcasted_iota(jnp.int32, (_BLK, _C), 0) + i * _BLK
    mask = (rows < _N).astype(jnp.float32)
    out = out * mask
    dinv = lax.rsqrt(deg_ref[...] + 1.0) * mask
    s = 0.9 * dinv * dinv
    outa = 0.1 * out
    zt0_ref[...] = dinv * out
    r0_ref[...] = s * out + outa
    outa_ref[...] = outa
    dv_ref[...] = dinv


def _prep(x_p, W1, b1, W2, b2, degrows):
    o16 = jax.ShapeDtypeStruct((_NP, _C), jnp.float32)
    bs16 = pl.BlockSpec((_BLK, _C), lambda i: (i, 0))
    return pl.pallas_call(
        _prep_body,
        grid=(_NP // _BLK,),
        in_specs=[
            pl.BlockSpec((_BLK, 128), lambda i: (i, 0)),
            pl.BlockSpec((128, 128), lambda i: (0, 0)),
            pl.BlockSpec((1, 128), lambda i: (0, 0)),
            pl.BlockSpec((_C, 128), lambda i: (0, 0)),
            pl.BlockSpec((1, _C), lambda i: (0, 0)),
            bs16,
        ],
        out_specs=[bs16] * 4,
        out_shape=[o16] * 4,
    )(x_p, W1, b1.reshape(1, 128), W2, b2.reshape(1, _C), degrows)


def _prop_body(zt0, r0, outah, dvh, edges,
               z_out,
               srcbuf, dstbuf, rows0, rows1, rows2, zb, outab, dvb, rb, accv,
               acc, ztS,
               sg0, sg1, sg2, ss0, ss1, ss2):
    t = lax.axis_index("s")
    sl = pl.ds(t * _NPT, _NPT)
    pltpu.sync_copy(edges.at[pl.ds(t * _EPT, _EPT)], srcbuf)
    _split_packed(srcbuf, dstbuf)
    pltpu.sync_copy(outah.at[sl], outab)
    pltpu.sync_copy(dvh.at[sl], dvb)
    pltpu.sync_copy(r0.at[sl], rb)
    pltpu.sync_copy(zt0.at[sl], ztS.at[sl])
    _zero_fill(zb)
    for j in range(_NPT // 128):
        pltpu.sync_copy(zb, acc.at[pl.ds(t * _NPT + j * 128, 128)])
    plsc.subcore_barrier()

    nch = _EPT // _CH
    bufs = (rows0, rows1, rows2)
    gsems = (sg0, sg1, sg2)
    ssems = (ss0, ss1, ss2)

    def gather(j):
        return pltpu.async_copy(
            ztS.at[srcbuf.at[pl.ds(j * _CH, _CH)]], bufs[j % 3],
            gsems[j % 3])

    def scat(j):
        return pltpu.async_copy(
            bufs[j % 3], acc.at[dstbuf.at[pl.ds(j * _CH, _CH)]], ssems[j % 3],
            add=True)

    for k in range(_K):
        last = k == _K - 1

        gd = {0: gather(0), 1: gather(1)}
        sd = [None, None, None]
        for j in range(nch):
            gd.pop(j).wait()
            sd[j % 3] = scat(j)
            nxt = j + 2
            if nxt < nch:
                if sd[nxt % 3] is not None:
                    sd[nxt % 3].wait()
                gd[nxt] = gather(nxt)
        for j in range(max(0, nch - 3), nch):
            sd[j % 3].wait()
        plsc.subcore_barrier()

        pltpu.sync_copy(acc.at[sl], accv)

        def cbody(i, _):
            a = accv[i]
            z16 = 0.9 * (dvb[i] * a) + rb[i]
            zt16 = dvb[i] * z16
            accv[i] = z16 if last else zt16
            rb[i] = 0.9 * (dvb[i] * zt16) + outab[i]
            return 0
        lax.fori_loop(0, _NPT, cbody, 0)
        if last:
            pltpu.sync_copy(accv, z_out.at[sl])
        else:
            pltpu.sync_copy(accv, ztS.at[sl])
            for j in range(_NPT // 128):
                pltpu.sync_copy(zb, acc.at[pl.ds(t * _NPT + j * 128, 128)])
        plsc.subcore_barrier()


def _propagate(zt0, r0, outa, dv, edges):
    o16 = jax.ShapeDtypeStruct((_NP, _C), jnp.float32)
    z = pl.kernel(
        _prop_body,
        out_type=o16,
        mesh=_mesh,
        compiler_params=_sc_params,
        scratch_types=[
            pltpu.VMEM((_EPT,), jnp.int32),
            pltpu.VMEM((_EPT,), jnp.int32),
            pltpu.VMEM((_CH, _C), jnp.float32),
            pltpu.VMEM((_CH, _C), jnp.float32),
            pltpu.VMEM((_CH, _C), jnp.float32),
            pltpu.VMEM((128, _C), jnp.float32),
            pltpu.VMEM((_NPT, _C), jnp.float32),
            pltpu.VMEM((_NPT, _C), jnp.float32),
            pltpu.VMEM((_NPT, _C), jnp.float32),
            pltpu.VMEM((_NPT, _C), jnp.float32),
            pltpu.VMEM_SHARED((_NP, _C), jnp.float32),
            pltpu.VMEM_SHARED((_NP, _C), jnp.float32),
            pltpu.SemaphoreType.DMA,
            pltpu.SemaphoreType.DMA,
            pltpu.SemaphoreType.DMA,
            pltpu.SemaphoreType.DMA,
            pltpu.SemaphoreType.DMA,
            pltpu.SemaphoreType.DMA,
        ],
    )(zt0, r0, outa, dv, edges)
    return z


def kernel(x, edge_index, W1, b1, W2, b2):
    pad = jnp.full((_EP - _E,), _N, jnp.int32)
    src = jnp.concatenate([edge_index[0], pad])
    dst = jnp.concatenate([edge_index[1], pad])
    edges = jnp.left_shift(src, 14) | dst
    x_p = jnp.pad(x, ((0, _NP - _N), (0, 0)))
    degrows = _degree(edges)
    zt0, r0, outa, dv = _prep(x_p, W1, b1, W2, b2, degrows)
    z = _propagate(zt0, r0, outa, dv, edges)
    return z[:_N]


# speedup vs baseline: 51.1597x; 1.0529x over previous
import jax
import jax.numpy as jnp
from jax import lax
from jax.experimental import pallas as pl
from jax.experimental.pallas import tpu as pltpu
from jax.experimental.pallas import tpu_sc as plsc

_N = 10000
_NP = 10240
_E = 320000
_EP = 327680
_C = 16
_K = 10
_NT = 16
_NPT = _NP // _NT
_EPT = _EP // _NT
_CH = 512

_mesh = plsc.VectorSubcoreMesh(
    core_axis_name="c", subcore_axis_name="s", num_cores=1, num_subcores=_NT)
_sc_params = pltpu.CompilerParams(use_tc_tiling_on_sc=False,
                                 needs_layout_passes=False)


def _zero_fill(buf):
    def body(i, _):
        buf[i] = jnp.zeros((16,), jnp.float32)
        return 0
    lax.fori_loop(0, buf.shape[0], body, 0)


def _split_packed(packed_buf, dst_buf):
    def body(i, _):
        v = packed_buf[pl.ds(i * 16, 16)]
        dst_buf[pl.ds(i * 16, 16)] = lax.bitwise_and(v, 16383)
        packed_buf[pl.ds(i * 16, 16)] = lax.shift_right_logical(v, 14)
        return 0
    lax.fori_loop(0, _EPT // 16, body, 0)


_BLK = 512


def _prep_body(x_ref, w1_ref, b1_ref, w2_ref, b2_ref, out_ref):
    i = pl.program_id(0)
    xb = x_ref[...]
    h = jnp.maximum(
        lax.dot_general(xb, w1_ref[...], (((1,), (1,)), ((), ())),
                        preferred_element_type=jnp.float32) + b1_ref[...], 0.0)
    out = lax.dot_general(h, w2_ref[...], (((1,), (1,)), ((), ())),
                          preferred_element_type=jnp.float32) + b2_ref[...]
    rows = lax.broadcasted_iota(jnp.int32, (_BLK, _C), 0) + i * _BLK
    mask = (rows < _N).astype(jnp.float32)
    out_ref[...] = out * mask


def _prep(x_p, W1, b1, W2, b2):
    return pl.pallas_call(
        _prep_body,
        grid=(_NP // _BLK,),
        in_specs=[
            pl.BlockSpec((_BLK, 128), lambda i: (i, 0)),
            pl.BlockSpec((128, 128), lambda i: (0, 0)),
            pl.BlockSpec((1, 128), lambda i: (0, 0)),
            pl.BlockSpec((_C, 128), lambda i: (0, 0)),
            pl.BlockSpec((1, _C), lambda i: (0, 0)),
        ],
        out_specs=pl.BlockSpec((_BLK, _C), lambda i: (i, 0)),
        out_shape=jax.ShapeDtypeStruct((_NP, _C), jnp.float32),
    )(x_p, W1, b1.reshape(1, 128), W2, b2.reshape(1, _C))


def _prop_body(outh, edges,
               z_out,
               srcbuf, dstbuf, rows0, rows1, rows2, zb, outab, dvb, rb, accv,
               acc, ztS,
               sg0, sg1, sg2, ss0, ss1, ss2):
    t = lax.axis_index("s")
    sl = pl.ds(t * _NPT, _NPT)
    nch = _EPT // _CH
    pltpu.sync_copy(edges.at[pl.ds(t * _EPT, _EPT)], srcbuf)
    _split_packed(srcbuf, dstbuf)
    _zero_fill(zb)
    for j in range(_NPT // 128):
        pltpu.sync_copy(zb, acc.at[pl.ds(t * _NPT + j * 128, 128)])

    def ofill(i, _):
        rows0[i] = jnp.full((16,), 1.0, jnp.float32)
        return 0
    lax.fori_loop(0, _CH, ofill, 0)
    plsc.subcore_barrier()

    def dbody(j, _):
        pltpu.sync_copy(rows0, acc.at[dstbuf.at[pl.ds(j * _CH, _CH)]],
                        add=True)
        return 0
    lax.fori_loop(0, nch, dbody, 0)
    plsc.subcore_barrier()

    pltpu.sync_copy(acc.at[sl], accv)

    def pbody(i, _):
        x = accv[i] + 1.0
        magic = jnp.full((16,), 0x5F3759DF, jnp.int32)
        one = jnp.full((16,), 1, jnp.int32)
        yi = magic - lax.shift_right_logical(plsc.bitcast(x, jnp.int32), one)
        y = plsc.bitcast(yi, jnp.float32)
        y = y * (1.5 - 0.5 * x * y * y)
        y = y * (1.5 - 0.5 * x * y * y)
        y = y * (1.5 - 0.5 * x * y * y)
        dvb[i] = y
        return 0
    lax.fori_loop(0, _NPT, pbody, 0)
    pltpu.sync_copy(outh.at[sl], accv)

    def qbody(i, _):
        o = accv[i]
        d = dvb[i]
        oa = 0.1 * o
        outab[i] = oa
        zt = d * o
        rb[i] = 0.9 * (d * zt) + oa
        accv[i] = zt
        return 0
    lax.fori_loop(0, _NPT, qbody, 0)
    pltpu.sync_copy(accv, ztS.at[sl])
    for j in range(_NPT // 128):
        pltpu.sync_copy(zb, acc.at[pl.ds(t * _NPT + j * 128, 128)])
    plsc.subcore_barrier()
    bufs = (rows0, rows1, rows2)
    gsems = (sg0, sg1, sg2)
    ssems = (ss0, ss1, ss2)

    def gather(j):
        return pltpu.async_copy(
            ztS.at[srcbuf.at[pl.ds(j * _CH, _CH)]], bufs[j % 3],
            gsems[j % 3])

    def scat(j):
        return pltpu.async_copy(
            bufs[j % 3], acc.at[dstbuf.at[pl.ds(j * _CH, _CH)]], ssems[j % 3],
            add=True)

    for k in range(_K):
        last = k == _K - 1

        gd = {0: gather(0), 1: gather(1)}
        sd = [None, None, None]
        for j in range(nch):
            gd.pop(j).wait()
            sd[j % 3] = scat(j)
            nxt = j + 2
            if nxt < nch:
                if sd[nxt % 3] is not None:
                    sd[nxt % 3].wait()
                gd[nxt] = gather(nxt)
        for j in range(max(0, nch - 3), nch):
            sd[j % 3].wait()
        plsc.subcore_barrier()

        pltpu.sync_copy(acc.at[sl], accv)

        def cbody(i, _):
            a = accv[i]
            z16 = 0.9 * (dvb[i] * a) + rb[i]
            zt16 = dvb[i] * z16
            accv[i] = z16 if last else zt16
            rb[i] = 0.9 * (dvb[i] * zt16) + outab[i]
            return 0
        lax.fori_loop(0, _NPT, cbody, 0)
        if last:
            pltpu.sync_copy(accv, z_out.at[sl])
        else:
            pltpu.sync_copy(accv, ztS.at[sl])
            for j in range(_NPT // 128):
                pltpu.sync_copy(zb, acc.at[pl.ds(t * _NPT + j * 128, 128)])
        plsc.subcore_barrier()


def _propagate(out, edges):
    o16 = jax.ShapeDtypeStruct((_NP, _C), jnp.float32)
    z = pl.kernel(
        _prop_body,
        out_type=o16,
        mesh=_mesh,
        compiler_params=_sc_params,
        scratch_types=[
            pltpu.VMEM((_EPT,), jnp.int32),
            pltpu.VMEM((_EPT,), jnp.int32),
            pltpu.VMEM((_CH, _C), jnp.float32),
            pltpu.VMEM((_CH, _C), jnp.float32),
            pltpu.VMEM((_CH, _C), jnp.float32),
            pltpu.VMEM((128, _C), jnp.float32),
            pltpu.VMEM((_NPT, _C), jnp.float32),
            pltpu.VMEM((_NPT, _C), jnp.float32),
            pltpu.VMEM((_NPT, _C), jnp.float32),
            pltpu.VMEM((_NPT, _C), jnp.float32),
            pltpu.VMEM_SHARED((_NP, _C), jnp.float32),
            pltpu.VMEM_SHARED((_NP, _C), jnp.float32),
            pltpu.SemaphoreType.DMA,
            pltpu.SemaphoreType.DMA,
            pltpu.SemaphoreType.DMA,
            pltpu.SemaphoreType.DMA,
            pltpu.SemaphoreType.DMA,
            pltpu.SemaphoreType.DMA,
        ],
    )(out, edges)
    return z


def kernel(x, edge_index, W1, b1, W2, b2):
    pad = jnp.full((_EP - _E,), _N, jnp.int32)
    src = jnp.concatenate([edge_index[0], pad])
    dst = jnp.concatenate([edge_index[1], pad])
    edges = jnp.left_shift(src, 14) | dst
    x_p = jnp.pad(x, ((0, _NP - _N), (0, 0)))
    out = _prep(x_p, W1, b1, W2, b2)
    z = _propagate(out, edges)
    return z[:_N]


# speedup vs baseline: 56.9625x; 1.1134x over previous
import jax
import jax.numpy as jnp
from jax import lax
from jax.experimental import pallas as pl
from jax.experimental.pallas import tpu as pltpu
from jax.experimental.pallas import tpu_sc as plsc

_N = 10000
_NP = 10240
_E = 320000
_EP = 327680
_C = 16
_K = 10
_NT = 16
_NPT = _NP // _NT
_EPT = _EP // _NT
_CH = 512

_mesh = plsc.VectorSubcoreMesh(
    core_axis_name="c", subcore_axis_name="s", num_cores=1, num_subcores=_NT)
_sc_params = pltpu.CompilerParams(use_tc_tiling_on_sc=False,
                                 needs_layout_passes=False)


def _zero_fill(buf):
    def body(i, _):
        buf[i] = jnp.zeros((16,), jnp.float32)
        return 0
    lax.fori_loop(0, buf.shape[0], body, 0)


def _split_packed(packed_buf, dst_buf):
    def body(i, _):
        v = packed_buf[pl.ds(i * 16, 16)]
        dst_buf[pl.ds(i * 16, 16)] = lax.bitwise_and(v, 16383)
        packed_buf[pl.ds(i * 16, 16)] = lax.shift_right_logical(v, 14)
        return 0
    lax.fori_loop(0, _EPT // 16, body, 0)


_BLK = 512


def _prep_body(x_ref, w1_ref, b1_ref, w2_ref, b2_ref, out_ref):
    i = pl.program_id(0)
    xb = x_ref[...]
    h = jnp.maximum(
        lax.dot_general(xb, w1_ref[...], (((1,), (1,)), ((), ())),
                        preferred_element_type=jnp.float32) + b1_ref[...], 0.0)
    out = lax.dot_general(h, w2_ref[...], (((1,), (1,)), ((), ())),
                          preferred_element_type=jnp.float32) + b2_ref[...]
    rows = lax.broadcasted_iota(jnp.int32, (_BLK, _C), 0) + i * _BLK
    mask = (rows < _N).astype(jnp.float32)
    out_ref[...] = out * mask


def _prep(x_p, W1, b1, W2, b2):
    return pl.pallas_call(
        _prep_body,
        grid=(_NP // _BLK,),
        in_specs=[
            pl.BlockSpec((_BLK, 128), lambda i: (i, 0)),
            pl.BlockSpec((128, 128), lambda i: (0, 0)),
            pl.BlockSpec((1, 128), lambda i: (0, 0)),
            pl.BlockSpec((_C, 128), lambda i: (0, 0)),
            pl.BlockSpec((1, _C), lambda i: (0, 0)),
        ],
        out_specs=pl.BlockSpec((_BLK, _C), lambda i: (i, 0)),
        out_shape=jax.ShapeDtypeStruct((_NP, _C), jnp.float32),
    )(x_p, W1, b1.reshape(1, 128), W2, b2.reshape(1, _C))


def _prop_body(outh, edges,
               z_out,
               srcbuf, dstbuf, rows0, rows1, rows2, zb, outab, dvb, rb, accv,
               acc, ztS,
               sg0, sg1, sg2, ss0, ss1, ss2, zsem):
    t = lax.axis_index("s")
    sl = pl.ds(t * _NPT, _NPT)
    nch = _EPT // _CH
    pltpu.sync_copy(edges.at[pl.ds(t * _EPT, _EPT)], srcbuf)
    _split_packed(srcbuf, dstbuf)
    _zero_fill(zb)
    for j in range(_NPT // 128):
        pltpu.sync_copy(zb, acc.at[pl.ds(t * _NPT + j * 128, 128)])

    def ofill(i, _):
        rows0[i] = jnp.full((16,), 1.0, jnp.float32)
        return 0
    lax.fori_loop(0, _CH, ofill, 0)
    plsc.subcore_barrier()

    def dbody(j, _):
        pltpu.sync_copy(rows0, acc.at[dstbuf.at[pl.ds(j * _CH, _CH)]],
                        add=True)
        return 0
    lax.fori_loop(0, nch, dbody, 0)
    plsc.subcore_barrier()

    pltpu.sync_copy(acc.at[sl], accv)

    def pbody(i, _):
        x = accv[i] + 1.0
        magic = jnp.full((16,), 0x5F3759DF, jnp.int32)
        one = jnp.full((16,), 1, jnp.int32)
        yi = magic - lax.shift_right_logical(plsc.bitcast(x, jnp.int32), one)
        y = plsc.bitcast(yi, jnp.float32)
        y = y * (1.5 - 0.5 * x * y * y)
        y = y * (1.5 - 0.5 * x * y * y)
        y = y * (1.5 - 0.5 * x * y * y)
        dvb[i] = y
        return 0
    lax.fori_loop(0, _NPT, pbody, 0)
    pltpu.sync_copy(outh.at[sl], accv)

    def qbody(i, _):
        o = accv[i]
        d = dvb[i]
        oa = 0.1 * o
        outab[i] = oa
        zt = d * o
        rb[i] = 0.9 * (d * zt) + oa
        accv[i] = zt
        return 0
    lax.fori_loop(0, _NPT, qbody, 0)
    pltpu.sync_copy(accv, ztS.at[sl])
    for j in range(_NPT // 128):
        pltpu.sync_copy(zb, acc.at[pl.ds(t * _NPT + j * 128, 128)])
    plsc.subcore_barrier()
    bufs = (rows0, rows1, rows2)
    gsems = (sg0, sg1, sg2)
    ssems = (ss0, ss1, ss2)

    def gather(j):
        return pltpu.async_copy(
            ztS.at[srcbuf.at[pl.ds(j * _CH, _CH)]], bufs[j % 3],
            gsems[j % 3])

    def scat(j):
        return pltpu.async_copy(
            bufs[j % 3], acc.at[dstbuf.at[pl.ds(j * _CH, _CH)]], ssems[j % 3],
            add=True)

    for k in range(_K):
        last = k == _K - 1

        gd = {0: gather(0), 1: gather(1)}
        sd = [None, None, None]
        for j in range(nch):
            gd.pop(j).wait()
            sd[j % 3] = scat(j)
            nxt = j + 2
            if nxt < nch:
                if sd[nxt % 3] is not None:
                    sd[nxt % 3].wait()
                gd[nxt] = gather(nxt)
        for j in range(max(0, nch - 3), nch):
            sd[j % 3].wait()
        plsc.subcore_barrier()

        pltpu.sync_copy(acc.at[sl], accv)
        zd = None
        if not last:
            zd = [pltpu.async_copy(
                      zb, acc.at[pl.ds(t * _NPT + j * 128, 128)], zsem)
                  for j in range(_NPT // 128)]

        def cbody(i, _):
            for u in range(4):
                r_ = i * 4 + u
                a = accv[r_]
                z16 = 0.9 * (dvb[r_] * a) + rb[r_]
                zt16 = dvb[r_] * z16
                accv[r_] = z16 if last else zt16
                rb[r_] = 0.9 * (dvb[r_] * zt16) + outab[r_]
            return 0
        lax.fori_loop(0, _NPT // 4, cbody, 0)
        if last:
            nr = jnp.minimum(_N - t * _NPT, _NPT)

            @pl.when(t * _NPT + _NPT <= _N)
            def _():
                pltpu.sync_copy(accv, z_out.at[pl.ds(t * _NPT, _NPT)])

            @pl.when(t * _NPT + _NPT > _N)
            def _():
                pltpu.sync_copy(accv.at[pl.ds(0, _N - 15 * _NPT)],
                                z_out.at[pl.ds(15 * _NPT, _N - 15 * _NPT)])
        else:
            pltpu.sync_copy(accv, ztS.at[sl])
            for d in zd:
                d.wait()
        plsc.subcore_barrier()


def _propagate(out, edges):
    o16 = jax.ShapeDtypeStruct((_N, _C), jnp.float32)
    z = pl.kernel(
        _prop_body,
        out_type=o16,
        mesh=_mesh,
        compiler_params=_sc_params,
        scratch_types=[
            pltpu.VMEM((_EPT,), jnp.int32),
            pltpu.VMEM((_EPT,), jnp.int32),
            pltpu.VMEM((_CH, _C), jnp.float32),
            pltpu.VMEM((_CH, _C), jnp.float32),
            pltpu.VMEM((_CH, _C), jnp.float32),
            pltpu.VMEM((128, _C), jnp.float32),
            pltpu.VMEM((_NPT, _C), jnp.float32),
            pltpu.VMEM((_NPT, _C), jnp.float32),
            pltpu.VMEM((_NPT, _C), jnp.float32),
            pltpu.VMEM((_NPT, _C), jnp.float32),
            pltpu.VMEM_SHARED((_NP, _C), jnp.float32),
            pltpu.VMEM_SHARED((_NP, _C), jnp.float32),
            pltpu.SemaphoreType.DMA,
            pltpu.SemaphoreType.DMA,
            pltpu.SemaphoreType.DMA,
            pltpu.SemaphoreType.DMA,
            pltpu.SemaphoreType.DMA,
            pltpu.SemaphoreType.DMA,
            pltpu.SemaphoreType.DMA,
        ],
    )(out, edges)
    return z


def kernel(x, edge_index, W1, b1, W2, b2):
    pad = jnp.full((_EP - _E,), _N, jnp.int32)
    src = jnp.concatenate([edge_index[0], pad])
    dst = jnp.concatenate([edge_index[1], pad])
    edges = jnp.left_shift(src, 14) | dst
    x_p = jnp.pad(x, ((0, _NP - _N), (0, 0)))
    out = _prep(x_p, W1, b1, W2, b2)
    return _propagate(out, edges)


# speedup vs baseline: 57.0183x; 1.0010x over previous
import jax
import jax.numpy as jnp
from jax import lax
from jax.experimental import pallas as pl
from jax.experimental.pallas import tpu as pltpu
from jax.experimental.pallas import tpu_sc as plsc

_N = 10000
_NP = 10240
_E = 320000
_EP = 327680
_C = 16
_K = 10
_NT = 16
_NPT = _NP // _NT
_EPT = _EP // _NT
_CH = 512

_mesh = plsc.VectorSubcoreMesh(
    core_axis_name="c", subcore_axis_name="s", num_cores=1, num_subcores=_NT)
_sc_params = pltpu.CompilerParams(use_tc_tiling_on_sc=False,
                                 needs_layout_passes=False)


def _zero_fill(buf):
    def body(i, _):
        buf[i] = jnp.zeros((16,), jnp.float32)
        return 0
    lax.fori_loop(0, buf.shape[0], body, 0)


def _split_packed(packed_buf, dst_buf):
    def body(i, _):
        v = packed_buf[pl.ds(i * 16, 16)]
        dst_buf[pl.ds(i * 16, 16)] = lax.bitwise_and(v, 16383)
        packed_buf[pl.ds(i * 16, 16)] = lax.shift_right_logical(v, 14)
        return 0
    lax.fori_loop(0, _EPT // 16, body, 0)


_BLK = 512


def _prep_body(x_ref, w1_ref, b1_ref, w2_ref, b2_ref, out_ref):
    i = pl.program_id(0)
    xb = x_ref[...]
    h = jnp.maximum(
        lax.dot_general(xb, w1_ref[...], (((1,), (1,)), ((), ())),
                        preferred_element_type=jnp.float32) + b1_ref[...], 0.0)
    out = lax.dot_general(h, w2_ref[...], (((1,), (1,)), ((), ())),
                          preferred_element_type=jnp.float32) + b2_ref[...]
    rows = lax.broadcasted_iota(jnp.int32, (_BLK, _C), 0) + i * _BLK
    mask = (rows < _N).astype(jnp.float32)
    out_ref[...] = out * mask


def _prep(x_p, W1, b1, W2, b2):
    return pl.pallas_call(
        _prep_body,
        grid=(_NP // _BLK,),
        in_specs=[
            pl.BlockSpec((_BLK, 128), lambda i: (i, 0)),
            pl.BlockSpec((128, 128), lambda i: (0, 0)),
            pl.BlockSpec((1, 128), lambda i: (0, 0)),
            pl.BlockSpec((_C, 128), lambda i: (0, 0)),
            pl.BlockSpec((1, _C), lambda i: (0, 0)),
        ],
        out_specs=pl.BlockSpec((_BLK, _C), lambda i: (i, 0)),
        out_shape=jax.ShapeDtypeStruct((_NP, _C), jnp.float32),
    )(x_p, W1, b1.reshape(1, 128), W2, b2.reshape(1, _C))


def _prop_body(outh, edges,
               z_out,
               srcbuf, dstbuf, rows0, rows1, rows2, zb, outab, dvb, rb, accv,
               acc, ztS,
               sg0, sg1, sg2, ss0, ss1, ss2, zsem):
    t = lax.axis_index("s")
    sl = pl.ds(t * _NPT, _NPT)
    nch = _EPT // _CH
    pltpu.sync_copy(edges.at[pl.ds(t * _EPT, _EPT)], srcbuf)
    _split_packed(srcbuf, dstbuf)
    _zero_fill(zb)
    for j in range(_NPT // 128):
        pltpu.sync_copy(zb, acc.at[pl.ds(t * _NPT + j * 128, 128)])

    def ofill(i, _):
        rows0[i] = jnp.full((16,), 1.0, jnp.float32)
        return 0
    lax.fori_loop(0, _CH, ofill, 0)
    plsc.subcore_barrier()

    dsems = (ss0, ss1)
    dds = [None, None]
    for j in range(nch):
        if dds[j % 2] is not None:
            dds[j % 2].wait()
        dds[j % 2] = pltpu.async_copy(
            rows0, acc.at[dstbuf.at[pl.ds(j * _CH, _CH)]], dsems[j % 2],
            add=True)
    dds[0].wait()
    dds[1].wait()
    plsc.subcore_barrier()

    pltpu.sync_copy(acc.at[sl], accv)

    def pbody(i, _):
        x = accv[i] + 1.0
        magic = jnp.full((16,), 0x5F3759DF, jnp.int32)
        one = jnp.full((16,), 1, jnp.int32)
        yi = magic - lax.shift_right_logical(plsc.bitcast(x, jnp.int32), one)
        y = plsc.bitcast(yi, jnp.float32)
        y = y * (1.5 - 0.5 * x * y * y)
        y = y * (1.5 - 0.5 * x * y * y)
        y = y * (1.5 - 0.5 * x * y * y)
        dvb[i] = y
        return 0
    lax.fori_loop(0, _NPT, pbody, 0)
    pltpu.sync_copy(outh.at[sl], accv)

    def qbody(i, _):
        o = accv[i]
        d = dvb[i]
        oa = 0.1 * o
        outab[i] = oa
        zt = d * o
        rb[i] = 0.9 * (d * zt) + oa
        accv[i] = zt
        return 0
    lax.fori_loop(0, _NPT, qbody, 0)
    pltpu.sync_copy(accv, ztS.at[sl])
    for j in range(_NPT // 128):
        pltpu.sync_copy(zb, acc.at[pl.ds(t * _NPT + j * 128, 128)])
    plsc.subcore_barrier()
    bufs = (rows0, rows1, rows2)
    gsems = (sg0, sg1, sg2)
    ssems = (ss0, ss1, ss2)

    def gather(j):
        return pltpu.async_copy(
            ztS.at[srcbuf.at[pl.ds(j * _CH, _CH)]], bufs[j % 3],
            gsems[j % 3])

    def scat(j):
        return pltpu.async_copy(
            bufs[j % 3], acc.at[dstbuf.at[pl.ds(j * _CH, _CH)]], ssems[j % 3],
            add=True)

    for k in range(_K):
        last = k == _K - 1

        gd = {0: gather(0), 1: gather(1)}
        sd = [None, None, None]
        for j in range(nch):
            gd.pop(j).wait()
            sd[j % 3] = scat(j)
            nxt = j + 2
            if nxt < nch:
                if sd[nxt % 3] is not None:
                    sd[nxt % 3].wait()
                gd[nxt] = gather(nxt)
        for j in range(max(0, nch - 3), nch):
            sd[j % 3].wait()
        plsc.subcore_barrier()

        pltpu.sync_copy(acc.at[sl], accv)
        zd = None
        if not last:
            zd = [pltpu.async_copy(
                      zb, acc.at[pl.ds(t * _NPT + j * 128, 128)], zsem)
                  for j in range(_NPT // 128)]

        def cbody(i, _):
            for u in range(4):
                r_ = i * 4 + u
                a = accv[r_]
                z16 = 0.9 * (dvb[r_] * a) + rb[r_]
                zt16 = dvb[r_] * z16
                accv[r_] = z16 if last else zt16
                rb[r_] = 0.9 * (dvb[r_] * zt16) + outab[r_]
            return 0
        lax.fori_loop(0, _NPT // 4, cbody, 0)
        if last:
            @pl.when(t * _NPT + _NPT <= _N)
            def _():
                pltpu.sync_copy(accv, z_out.at[pl.ds(t * _NPT, _NPT)])

            @pl.when(t * _NPT + _NPT > _N)
            def _():
                pltpu.sync_copy(accv.at[pl.ds(0, _N - 15 * _NPT)],
                                z_out.at[pl.ds(15 * _NPT, _N - 15 * _NPT)])
        else:
            pltpu.sync_copy(accv, ztS.at[sl])
            for d in zd:
                d.wait()
        plsc.subcore_barrier()


def _propagate(out, edges):
    o16 = jax.ShapeDtypeStruct((_N, _C), jnp.float32)
    z = pl.kernel(
        _prop_body,
        out_type=o16,
        mesh=_mesh,
        compiler_params=_sc_params,
        scratch_types=[
            pltpu.VMEM((_EPT,), jnp.int32),
            pltpu.VMEM((_EPT,), jnp.int32),
            pltpu.VMEM((_CH, _C), jnp.float32),
            pltpu.VMEM((_CH, _C), jnp.float32),
            pltpu.VMEM((_CH, _C), jnp.float32),
            pltpu.VMEM((128, _C), jnp.float32),
            pltpu.VMEM((_NPT, _C), jnp.float32),
            pltpu.VMEM((_NPT, _C), jnp.float32),
            pltpu.VMEM((_NPT, _C), jnp.float32),
            pltpu.VMEM((_NPT, _C), jnp.float32),
            pltpu.VMEM_SHARED((_NP, _C), jnp.float32),
            pltpu.VMEM_SHARED((_NP, _C), jnp.float32),
            pltpu.SemaphoreType.DMA,
            pltpu.SemaphoreType.DMA,
            pltpu.SemaphoreType.DMA,
            pltpu.SemaphoreType.DMA,
            pltpu.SemaphoreType.DMA,
            pltpu.SemaphoreType.DMA,
            pltpu.SemaphoreType.DMA,
        ],
    )(out, edges)
    return z


def kernel(x, edge_index, W1, b1, W2, b2):
    pad = jnp.full((_EP - _E,), _N, jnp.int32)
    src = jnp.concatenate([edge_index[0], pad])
    dst = jnp.concatenate([edge_index[1], pad])
    edges = jnp.left_shift(src, 14) | dst
    x_p = jnp.pad(x, ((0, _NP - _N), (0, 0)))
    out = _prep(x_p, W1, b1, W2, b2)
    return _propagate(out, edges)
